# Initial kernel scaffold; baseline (speedup 1.0000x reference)
import jax
import jax.numpy as jnp
from jax.experimental import pallas as pl


def kernel(obs_pos, obs_mask, table):
    raise NotImplementedError("write your pallas kernel here")


# speedup vs baseline: 1.7599x; 1.7599x over previous
import functools

import jax
import jax.numpy as jnp
from jax import lax
from jax.experimental import pallas as pl
from jax.experimental.pallas import tpu as pltpu
from jax.experimental.pallas import tpu_sc as plsc


@functools.lru_cache(maxsize=None)
def _build(N: int, V: int, W: int):
    info = plsc.get_sparse_core_info()
    NC, NS, LANES = info.num_cores, info.num_subcores, info.num_lanes
    NW = NC * NS
    assert N % NW == 0
    b_per_w = N // NW
    CHUNK = 64
    assert b_per_w % CHUNK == 0
    n_chunks = b_per_w // CHUNK
    groups = W // LANES

    mesh = plsc.VectorSubcoreMesh(core_axis_name="c", subcore_axis_name="s")

    @functools.partial(
        pl.kernel,
        mesh=mesh,
        out_type=jax.ShapeDtypeStruct((N, W), jnp.float32),
        scratch_types=[
            pltpu.VMEM((b_per_w,), jnp.int32),
            pltpu.VMEM((b_per_w,), jnp.float32),
            pltpu.VMEM((CHUNK, W), jnp.float32),
        ],
    )
    def gather_mul(idx_hbm, mask_hbm, table_hbm, out_hbm, idx_v, mask_v, rows_v):
        wid = lax.axis_index("s") * NC + lax.axis_index("c")
        base = wid * b_per_w
        pltpu.sync_copy(idx_hbm.at[pl.ds(base, b_per_w)], idx_v)
        pltpu.sync_copy(mask_hbm.at[pl.ds(base, b_per_w)], mask_v)

        def chunk_body(c, _):
            off = c * CHUNK
            pltpu.sync_copy(table_hbm.at[idx_v.at[pl.ds(off, CHUNK)]], rows_v)

            def rg_body(rg, _):
                mvec16 = mask_v[pl.ds(off + rg * LANES, LANES)]
                for j in range(LANES):
                    m_j = jnp.take(mvec16, jnp.full((LANES,), j, dtype=jnp.int32))
                    r = rg * LANES + j
                    for g in range(groups):
                        sl = pl.ds(g * LANES, LANES)
                        rows_v[r, sl] = rows_v[r, sl] * m_j
                return 0

            lax.fori_loop(0, CHUNK // LANES, rg_body, 0)
            pltpu.sync_copy(rows_v, out_hbm.at[pl.ds(base + off, CHUNK)])
            return 0

        lax.fori_loop(0, n_chunks, chunk_body, 0)

    return gather_mul


def kernel(obs_pos, obs_mask, table):
    B, L = obs_pos.shape
    V, W = table.shape
    N = B * L
    idx = obs_pos.reshape(N).astype(jnp.int32)
    mask = obs_mask.astype(jnp.float32).reshape(N)
    out = _build(N, V, W)(idx, mask, table)
    return out.reshape(B, L, W)


# speedup vs baseline: 2.4960x; 1.4183x over previous
import functools

import jax
import jax.numpy as jnp
from jax import lax
from jax.experimental import pallas as pl
from jax.experimental.pallas import tpu as pltpu
from jax.experimental.pallas import tpu_sc as plsc

CHUNK = 32
NBUF = 4


@functools.lru_cache(maxsize=None)
def _build(N: int, V: int, W: int):
    info = plsc.get_sparse_core_info()
    NC, NS, LANES = info.num_cores, info.num_subcores, info.num_lanes
    NW = NC * NS
    assert N % NW == 0
    b_per_w = N // NW
    assert b_per_w % (CHUNK * NBUF) == 0
    n_chunks = b_per_w // CHUNK
    n_outer = n_chunks // NBUF
    groups = W // LANES

    mesh = plsc.VectorSubcoreMesh(core_axis_name="c", subcore_axis_name="s")

    @functools.partial(
        pl.kernel,
        mesh=mesh,
        out_type=jax.ShapeDtypeStruct((N, W), jnp.float32),
        scratch_types=[
            pltpu.VMEM((b_per_w,), jnp.int32),
            pltpu.VMEM((b_per_w,), jnp.float32),
        ]
        + [pltpu.VMEM((CHUNK, W), jnp.float32)] * NBUF
        + [pltpu.SemaphoreType.DMA] * (2 * NBUF),
    )
    def gather_mul(idx_hbm, mask_hbm, table_hbm, out_hbm,
                   idx_v, mask_v, r0, r1, r2, r3,
                   g0, g1, g2, g3, s0, s1, s2, s3):
        rows = (r0, r1, r2, r3)
        gsem = (g0, g1, g2, g3)
        ssem = (s0, s1, s2, s3)
        wid = lax.axis_index("s") * NC + lax.axis_index("c")
        base = wid * b_per_w
        pltpu.sync_copy(idx_hbm.at[pl.ds(base, b_per_w)], idx_v)
        pltpu.sync_copy(mask_hbm.at[pl.ds(base, b_per_w)], mask_v)

        def gather_start(c, b):
            pltpu.async_copy(
                table_hbm.at[idx_v.at[pl.ds(c * CHUNK, CHUNK)]], rows[b], gsem[b])

        def gather_wait(b):
            pltpu.make_async_copy(
                table_hbm.at[idx_v.at[pl.ds(0, CHUNK)]], rows[b], gsem[b]).wait()

        def scatter_start(c, b):
            pltpu.async_copy(
                rows[b], out_hbm.at[pl.ds(base + c * CHUNK, CHUNK)], ssem[b])

        def scatter_wait(b):
            pltpu.make_async_copy(
                rows[b], out_hbm.at[pl.ds(base, CHUNK)], ssem[b]).wait()

        def maybe_mul(c, b):
            off = c * CHUNK
            mn = mask_v[pl.ds(off, LANES)]
            mx = mn
            for rg in range(1, CHUNK // LANES):
                mv = mask_v[pl.ds(off + rg * LANES, LANES)]
                mn = jnp.minimum(mn, mv)
                mx = jnp.maximum(mx, mv)
            lane = lax.iota(jnp.int32, LANES)
            for sh in (8, 4, 2, 1):
                perm = jnp.bitwise_and(lane + sh, LANES - 1)
                mn = jnp.minimum(mn, jnp.take(mn, perm))
                mx = jnp.maximum(mx, jnp.take(mx, perm))
            allones = jnp.logical_and(mn[0] == 1.0, mx[0] == 1.0)

            @pl.when(jnp.logical_not(allones))
            def _():
                def rg_body(rg, _):
                    mvec16 = mask_v[pl.ds(off + rg * LANES, LANES)]

                    def j_body(j, _):
                        m_j = jnp.take(mvec16, jnp.full((LANES,), j, jnp.int32))
                        r = rg * LANES + j
                        for g in range(groups):
                            sl = pl.ds(g * LANES, LANES)
                            rows[b][r, sl] = rows[b][r, sl] * m_j
                        return 0

                    lax.fori_loop(0, LANES, j_body, 0)
                    return 0

                lax.fori_loop(0, CHUNK // LANES, rg_body, 0)

        for b in range(NBUF):
            gather_start(b, b)

        def body(cp, _):
            for b in range(NBUF):
                c = cp * NBUF + b
                gather_wait(b)
                maybe_mul(c, b)
                scatter_start(c, b)
                b2 = (b + 2) % NBUF
                cond = (cp >= 1) if b < 2 else (cp < n_outer - 1)

                @pl.when(cond)
                def _(b2=b2, c=c):
                    scatter_wait(b2)
                    gather_start(c + 2, b2)

            return 0

        lax.fori_loop(0, n_outer, body, 0)
        for b in range(NBUF):
            scatter_wait(b)

    return gather_mul


def kernel(obs_pos, obs_mask, table):
    B, L = obs_pos.shape
    V, W = table.shape
    N = B * L
    idx = obs_pos.reshape(N).astype(jnp.int32)
    mask = obs_mask.astype(jnp.float32).reshape(N)
    out = _build(N, V, W)(idx, mask, table)
    return out.reshape(B, L, W)
